# baseline (speedup 1.0000x reference)
import functools

import jax
import jax.numpy as jnp
from jax import lax
from jax.experimental import pallas as pl
from jax.experimental.pallas import tpu as pltpu
from jax.experimental.pallas import tpu_sc as plsc

K_SEG = 1024
L = 16
SLICE = 6272


@functools.partial(jax.jit, static_argnums=(2, 3, 4))
def _sup_pix_pool(img, spx, B, C, HW):
    NC, NS = 2, 16
    NW = NC * NS
    G = NW // B
    CPG = C // G
    NSL = HW // SLICE
    VPS = SLICE // (2 * L)

    mesh = plsc.VectorSubcoreMesh(core_axis_name="c", subcore_axis_name="s")

    @functools.partial(
        pl.kernel,
        out_type=jax.ShapeDtypeStruct((B, C, K_SEG), jnp.float32),
        mesh=mesh,
        scratch_types=[
            pltpu.VMEM((HW,), jnp.int32),
            pltpu.VMEM((SLICE,), jnp.float32),
            pltpu.VMEM((SLICE,), jnp.float32),
            pltpu.VMEM((L, K_SEG), jnp.float32),
            pltpu.VMEM((L, K_SEG), jnp.float32),
            pltpu.VMEM((K_SEG,), jnp.float32),
            pltpu.SemaphoreType.DMA,
            pltpu.SemaphoreType.DMA,
        ],
        compiler_params=pltpu.CompilerParams(needs_layout_passes=False),
    )
    def pool(img_hbm, spx_hbm, out_hbm, lbl_ref, buf_a, buf_b,
             acc_a, acc_b, row_ref, sem_a, sem_b):
        wid = lax.axis_index("s") * NC + lax.axis_index("c")
        b = wid // G
        g = wid % G

        pltpu.sync_copy(spx_hbm.at[b], lbl_ref)
        lanes = lax.iota(jnp.int32, L)
        neg_inf = jnp.full((L,), -jnp.inf, dtype=jnp.float32)

        def init(v, carry):
            sl = pl.ds(v * L, L)
            for r in range(L):
                acc_a[r, sl] = neg_inf
                acc_b[r, sl] = neg_inf
            return carry

        lax.fori_loop(0, K_SEG // L, init, 0)

        def scatter_slice(buf, s):
            base = s * SLICE

            def body(v, carry):
                off = v * (2 * L)
                l0 = lbl_ref[pl.ds(base + off, L)]
                d0 = buf[pl.ds(off, L)]
                g0 = plsc.load_gather(acc_a, [lanes, l0])
                plsc.store_scatter(acc_a, [lanes, l0], jnp.maximum(g0, d0))
                l1 = lbl_ref[pl.ds(base + off + L, L)]
                d1 = buf[pl.ds(off + L, L)]
                g1 = plsc.load_gather(acc_b, [lanes, l1])
                plsc.store_scatter(acc_b, [lanes, l1], jnp.maximum(g1, d1))
                return carry

            lax.fori_loop(0, VPS, body, 0)

        def per_channel(j, carry):
            ch = g * CPG + j
            pltpu.async_copy(img_hbm.at[b, ch, pl.ds(0, SLICE)], buf_a, sem_a)

            def pair(p, c):
                s0 = 2 * p
                pltpu.make_async_copy(
                    img_hbm.at[b, ch, pl.ds(0, SLICE)], buf_a, sem_a).wait()
                pltpu.async_copy(
                    img_hbm.at[b, ch, pl.ds((s0 + 1) * SLICE, SLICE)],
                    buf_b, sem_b)
                scatter_slice(buf_a, s0)
                pltpu.make_async_copy(
                    img_hbm.at[b, ch, pl.ds(0, SLICE)], buf_b, sem_b).wait()

                @pl.when(p < NSL // 2 - 1)
                def _():
                    pltpu.async_copy(
                        img_hbm.at[b, ch, pl.ds((s0 + 2) * SLICE, SLICE)],
                        buf_a, sem_a)

                scatter_slice(buf_b, s0 + 1)
                return c

            lax.fori_loop(0, NSL // 2, pair, 0)

            def red(kv, c):
                sl = pl.ds(kv * L, L)
                vals = []
                for r in range(L):
                    vals.append(acc_a[r, sl])
                    vals.append(acc_b[r, sl])
                for r in range(L):
                    acc_a[r, sl] = neg_inf
                    acc_b[r, sl] = neg_inf
                while len(vals) > 1:
                    vals = [jnp.maximum(vals[i], vals[i + 1])
                            for i in range(0, len(vals) - 1, 2)] + \
                           ([vals[-1]] if len(vals) % 2 else [])
                row_ref[sl] = vals[0]
                return c

            lax.fori_loop(0, K_SEG // L, red, 0)
            pltpu.sync_copy(row_ref, out_hbm.at[b, ch])
            return carry

        lax.fori_loop(0, CPG, per_channel, 0)

    return pool(img, spx)


def kernel(img, spx):
    B, C, H, W = img.shape
    HW = H * W
    img2 = img.reshape(B, C, HW)
    spx2 = spx.reshape(B, HW).astype(jnp.int32)
    return _sup_pix_pool(img2, spx2, B, C, HW)


# speedup vs baseline: 1.2566x; 1.2566x over previous
import functools

import jax
import jax.numpy as jnp
from jax import lax
from jax.experimental import pallas as pl
from jax.experimental.pallas import tpu as pltpu
from jax.experimental.pallas import tpu_sc as plsc

K_SEG = 1024
L = 16
SLICE = 6272


@functools.partial(jax.jit, static_argnums=(2, 3, 4))
def _sup_pix_pool(img, spx, B, C, HW):
    NC, NS = 2, 16
    NW = NC * NS
    G = NW // B
    CPG = C // G
    NSL = HW // SLICE
    VPS = SLICE // (2 * L)

    mesh = plsc.VectorSubcoreMesh(core_axis_name="c", subcore_axis_name="s")

    @functools.partial(
        pl.kernel,
        out_type=jax.ShapeDtypeStruct((B, C, K_SEG), jnp.float32),
        mesh=mesh,
        scratch_types=[
            pltpu.VMEM((HW,), jnp.int32),
            pltpu.VMEM((SLICE,), jnp.float32),
            pltpu.VMEM((SLICE,), jnp.float32),
            pltpu.VMEM((L * K_SEG,), jnp.float32),
            pltpu.VMEM((L * K_SEG,), jnp.float32),
            pltpu.VMEM((K_SEG,), jnp.float32),
            pltpu.SemaphoreType.DMA,
            pltpu.SemaphoreType.DMA,
        ],
        compiler_params=pltpu.CompilerParams(needs_layout_passes=False),
    )
    def pool(img_hbm, spx_hbm, out_hbm, idx_ref, buf_a, buf_b,
             acc_a, acc_b, row_ref, sem_a, sem_b):
        wid = lax.axis_index("s") * NC + lax.axis_index("c")
        b = wid // G
        g = wid % G

        pltpu.sync_copy(spx_hbm.at[b], idx_ref)
        lane_off = lax.iota(jnp.int32, L) * K_SEG
        neg_inf = jnp.full((L,), -jnp.inf, dtype=jnp.float32)

        def mk_idx(v, carry):
            sl = pl.ds(v * L, L)
            idx_ref[sl] = idx_ref[sl] + lane_off
            return carry

        lax.fori_loop(0, HW // L, mk_idx, 0)

        def init(v, carry):
            sl = pl.ds(v * L, L)
            acc_a[sl] = neg_inf
            acc_b[sl] = neg_inf
            return carry

        lax.fori_loop(0, L * K_SEG // L, init, 0)

        def scatter_slice(buf, s):
            base = s * SLICE

            def body(v, carry):
                off = v * (2 * L)
                i0 = idx_ref[pl.ds(base + off, L)]
                d0 = buf[pl.ds(off, L)]
                g0 = plsc.load_gather(acc_a, [i0])
                plsc.store_scatter(acc_a, [i0], jnp.maximum(g0, d0))
                i1 = idx_ref[pl.ds(base + off + L, L)]
                d1 = buf[pl.ds(off + L, L)]
                g1 = plsc.load_gather(acc_b, [i1])
                plsc.store_scatter(acc_b, [i1], jnp.maximum(g1, d1))
                return carry

            lax.fori_loop(0, VPS, body, 0)

        def per_channel(j, carry):
            ch = g * CPG + j
            pltpu.async_copy(img_hbm.at[b, ch, pl.ds(0, SLICE)], buf_a, sem_a)

            def pair(p, c):
                s0 = 2 * p
                pltpu.make_async_copy(
                    img_hbm.at[b, ch, pl.ds(0, SLICE)], buf_a, sem_a).wait()
                pltpu.async_copy(
                    img_hbm.at[b, ch, pl.ds((s0 + 1) * SLICE, SLICE)],
                    buf_b, sem_b)
                scatter_slice(buf_a, s0)
                pltpu.make_async_copy(
                    img_hbm.at[b, ch, pl.ds(0, SLICE)], buf_b, sem_b).wait()

                @pl.when(p < NSL // 2 - 1)
                def _():
                    pltpu.async_copy(
                        img_hbm.at[b, ch, pl.ds((s0 + 2) * SLICE, SLICE)],
                        buf_a, sem_a)

                scatter_slice(buf_b, s0 + 1)
                return c

            lax.fori_loop(0, NSL // 2, pair, 0)

            def red(kv, c):
                sl = pl.ds(kv * L, L)
                vals = []
                for r in range(L):
                    rsl = pl.ds(r * K_SEG + kv * L, L)
                    vals.append(acc_a[rsl])
                    vals.append(acc_b[rsl])
                for r in range(L):
                    rsl = pl.ds(r * K_SEG + kv * L, L)
                    acc_a[rsl] = neg_inf
                    acc_b[rsl] = neg_inf
                while len(vals) > 1:
                    vals = [jnp.maximum(vals[i], vals[i + 1])
                            for i in range(0, len(vals) - 1, 2)] + \
                           ([vals[-1]] if len(vals) % 2 else [])
                row_ref[sl] = vals[0]
                return c

            lax.fori_loop(0, K_SEG // L, red, 0)
            pltpu.sync_copy(row_ref, out_hbm.at[b, ch])
            return carry

        lax.fori_loop(0, CPG, per_channel, 0)

    return pool(img, spx)


def kernel(img, spx):
    B, C, H, W = img.shape
    HW = H * W
    img2 = img.reshape(B, C, HW)
    spx2 = spx.reshape(B, HW).astype(jnp.int32)
    return _sup_pix_pool(img2, spx2, B, C, HW)


# speedup vs baseline: 1.6698x; 1.3288x over previous
import functools

import jax
import jax.numpy as jnp
from jax import lax
from jax.experimental import pallas as pl
from jax.experimental.pallas import tpu as pltpu
from jax.experimental.pallas import tpu_sc as plsc

K_SEG = 1024
L = 16
U = 4
SLICE = 3584


@functools.partial(jax.jit, static_argnums=(2, 3, 4))
def _sup_pix_pool(img, spx, B, C, HW):
    NC, NS = 2, 16
    NW = NC * NS
    G = NW // B
    CPG = C // G
    NSL = HW // SLICE
    VPS = SLICE // (U * L)

    mesh = plsc.VectorSubcoreMesh(core_axis_name="c", subcore_axis_name="s")

    acc_types = [pltpu.VMEM((L * K_SEG,), jnp.float32) for _ in range(U)]

    @functools.partial(
        pl.kernel,
        out_type=jax.ShapeDtypeStruct((B, C, K_SEG), jnp.float32),
        mesh=mesh,
        scratch_types=[
            pltpu.VMEM((HW,), jnp.int32),
            pltpu.VMEM((SLICE,), jnp.float32),
            pltpu.VMEM((SLICE,), jnp.float32),
            *acc_types,
            pltpu.VMEM((K_SEG,), jnp.float32),
            pltpu.SemaphoreType.DMA,
            pltpu.SemaphoreType.DMA,
        ],
        compiler_params=pltpu.CompilerParams(needs_layout_passes=False),
    )
    def pool(img_hbm, spx_hbm, out_hbm, idx_ref, buf_a, buf_b,
             *rest):
        accs = rest[:U]
        row_ref, sem_a, sem_b = rest[U], rest[U + 1], rest[U + 2]
        wid = lax.axis_index("s") * NC + lax.axis_index("c")
        b = wid // G
        g = wid % G

        pltpu.sync_copy(spx_hbm.at[b], idx_ref)
        lane_off = lax.iota(jnp.int32, L) * K_SEG
        neg_inf = jnp.full((L,), -jnp.inf, dtype=jnp.float32)

        def mk_idx(v, carry):
            sl = pl.ds(v * L, L)
            idx_ref[sl] = idx_ref[sl] + lane_off
            return carry

        lax.fori_loop(0, HW // L, mk_idx, 0)

        def init(v, carry):
            sl = pl.ds(v * L, L)
            for acc in accs:
                acc[sl] = neg_inf
            return carry

        lax.fori_loop(0, K_SEG, init, 0)

        def scatter_slice(buf, s):
            base = s * SLICE

            def body(v, carry):
                off = v * (U * L)
                ivecs = [idx_ref[pl.ds(base + off + u * L, L)]
                         for u in range(U)]
                dvecs = [buf[pl.ds(off + u * L, L)] for u in range(U)]
                for u in range(U):
                    gv = plsc.load_gather(accs[u], [ivecs[u]])
                    plsc.store_scatter(accs[u], [ivecs[u]],
                                       jnp.maximum(gv, dvecs[u]))
                return carry

            lax.fori_loop(0, VPS, body, 0)

        def per_channel(j, carry):
            ch = g * CPG + j
            pltpu.async_copy(img_hbm.at[b, ch, pl.ds(0, SLICE)], buf_a, sem_a)

            def pair(p, c):
                s0 = 2 * p
                pltpu.make_async_copy(
                    img_hbm.at[b, ch, pl.ds(0, SLICE)], buf_a, sem_a).wait()
                pltpu.async_copy(
                    img_hbm.at[b, ch, pl.ds((s0 + 1) * SLICE, SLICE)],
                    buf_b, sem_b)
                scatter_slice(buf_a, s0)
                pltpu.make_async_copy(
                    img_hbm.at[b, ch, pl.ds(0, SLICE)], buf_b, sem_b).wait()

                @pl.when(p < NSL // 2 - 1)
                def _():
                    pltpu.async_copy(
                        img_hbm.at[b, ch, pl.ds((s0 + 2) * SLICE, SLICE)],
                        buf_a, sem_a)

                scatter_slice(buf_b, s0 + 1)
                return c

            lax.fori_loop(0, NSL // 2, pair, 0)

            def red(kv, c):
                vals = []
                for r in range(L):
                    rsl = pl.ds(r * K_SEG + kv * L, L)
                    for acc in accs:
                        vals.append(acc[rsl])
                for r in range(L):
                    rsl = pl.ds(r * K_SEG + kv * L, L)
                    for acc in accs:
                        acc[rsl] = neg_inf
                while len(vals) > 1:
                    vals = [jnp.maximum(vals[i], vals[i + 1])
                            for i in range(0, len(vals) - 1, 2)] + \
                           ([vals[-1]] if len(vals) % 2 else [])
                row_ref[pl.ds(kv * L, L)] = vals[0]
                return c

            lax.fori_loop(0, K_SEG // L, red, 0)
            pltpu.sync_copy(row_ref, out_hbm.at[b, ch])
            return carry

        lax.fori_loop(0, CPG, per_channel, 0)

    return pool(img, spx)


def kernel(img, spx):
    B, C, H, W = img.shape
    HW = H * W
    img2 = img.reshape(B, C, HW)
    spx2 = spx.reshape(B, HW).astype(jnp.int32)
    return _sup_pix_pool(img2, spx2, B, C, HW)


# speedup vs baseline: 2.3127x; 1.3851x over previous
import functools

import jax
import jax.numpy as jnp
from jax import lax
from jax.experimental import pallas as pl
from jax.experimental.pallas import tpu as pltpu
from jax.experimental.pallas import tpu_sc as plsc

K_SEG = 1024
L = 16
U = 4
SLICE = 3584


@functools.partial(jax.jit, static_argnums=(2, 3, 4))
def _sup_pix_pool(img, spx, B, C, HW):
    NC, NS = 2, 16
    NW = NC * NS
    G = NW // B
    CPG = C // G
    NSL = HW // SLICE
    VPS = SLICE // (U * L)

    mesh = plsc.VectorSubcoreMesh(core_axis_name="c", subcore_axis_name="s")

    acc_types = [pltpu.VMEM((L * K_SEG,), jnp.float32) for _ in range(U)]

    @functools.partial(
        pl.kernel,
        out_type=jax.ShapeDtypeStruct((B, C, K_SEG), jnp.float32),
        mesh=mesh,
        scratch_types=[
            pltpu.VMEM((HW,), jnp.int32),
            pltpu.VMEM((SLICE,), jnp.float32),
            pltpu.VMEM((SLICE,), jnp.float32),
            *acc_types,
            pltpu.VMEM((K_SEG,), jnp.float32),
            pltpu.SemaphoreType.DMA,
            pltpu.SemaphoreType.DMA,
        ],
        compiler_params=pltpu.CompilerParams(needs_layout_passes=False),
    )
    def pool(img_hbm, spx_hbm, out_hbm, idx_ref, buf_a, buf_b,
             *rest):
        accs = rest[:U]
        row_ref, sem_a, sem_b = rest[U], rest[U + 1], rest[U + 2]
        wid = lax.axis_index("s") * NC + lax.axis_index("c")
        b = wid // G
        g = wid % G

        pltpu.sync_copy(spx_hbm.at[b], idx_ref)
        lane_off = lax.iota(jnp.int32, L) * K_SEG
        neg_inf = jnp.full((L,), -jnp.inf, dtype=jnp.float32)

        def mk_idx(v, carry):
            sl = pl.ds(v * L, L)
            idx_ref[sl] = idx_ref[sl] + lane_off
            return carry

        lax.fori_loop(0, HW // L, mk_idx, 0)

        def init(v, carry):
            sl = pl.ds(v * L, L)
            for acc in accs:
                acc[sl] = neg_inf
            return carry

        lax.fori_loop(0, K_SEG, init, 0)

        def scatter_slice(buf, s):
            base = s * SLICE

            def body(v, carry):
                off = v * (U * L)
                ivecs = [idx_ref[pl.ds(base + off + u * L, L)]
                         for u in range(U)]
                dvecs = [buf[pl.ds(off + u * L, L)] for u in range(U)]
                gvs = [plsc.load_gather(accs[u], [ivecs[u]])
                       for u in range(U)]
                for u in range(U):
                    plsc.store_scatter(accs[u], [ivecs[u]],
                                       jnp.maximum(gvs[u], dvecs[u]))
                return carry

            lax.fori_loop(0, VPS, body, 0)

        def per_channel(j, carry):
            ch = g * CPG + j
            pltpu.async_copy(img_hbm.at[b, ch, pl.ds(0, SLICE)], buf_a, sem_a)

            def pair(p, c):
                s0 = 2 * p
                pltpu.make_async_copy(
                    img_hbm.at[b, ch, pl.ds(0, SLICE)], buf_a, sem_a).wait()
                pltpu.async_copy(
                    img_hbm.at[b, ch, pl.ds((s0 + 1) * SLICE, SLICE)],
                    buf_b, sem_b)
                scatter_slice(buf_a, s0)
                pltpu.make_async_copy(
                    img_hbm.at[b, ch, pl.ds(0, SLICE)], buf_b, sem_b).wait()

                @pl.when(p < NSL // 2 - 1)
                def _():
                    pltpu.async_copy(
                        img_hbm.at[b, ch, pl.ds((s0 + 2) * SLICE, SLICE)],
                        buf_a, sem_a)

                scatter_slice(buf_b, s0 + 1)
                return c

            lax.fori_loop(0, NSL // 2, pair, 0)

            def red(kv, c):
                vals = []
                for r in range(L):
                    rsl = pl.ds(r * K_SEG + kv * L, L)
                    for acc in accs:
                        vals.append(acc[rsl])
                for r in range(L):
                    rsl = pl.ds(r * K_SEG + kv * L, L)
                    for acc in accs:
                        acc[rsl] = neg_inf
                while len(vals) > 1:
                    vals = [jnp.maximum(vals[i], vals[i + 1])
                            for i in range(0, len(vals) - 1, 2)] + \
                           ([vals[-1]] if len(vals) % 2 else [])
                row_ref[pl.ds(kv * L, L)] = vals[0]
                return c

            lax.fori_loop(0, K_SEG // L, red, 0)
            pltpu.sync_copy(row_ref, out_hbm.at[b, ch])
            return carry

        lax.fori_loop(0, CPG, per_channel, 0)

    return pool(img, spx)


def kernel(img, spx):
    B, C, H, W = img.shape
    HW = H * W
    img2 = img.reshape(B, C, HW)
    spx2 = spx.reshape(B, HW).astype(jnp.int32)
    return _sup_pix_pool(img2, spx2, B, C, HW)


# speedup vs baseline: 2.4497x; 1.0592x over previous
import functools

import jax
import jax.numpy as jnp
from jax import lax
from jax.experimental import pallas as pl
from jax.experimental.pallas import tpu as pltpu
from jax.experimental.pallas import tpu_sc as plsc

K_SEG = 1024
L = 16
CH = 4
SLICE = 1792


@functools.partial(jax.jit, static_argnums=(2, 3, 4))
def _sup_pix_pool(img, spx, B, C, HW):
    NC, NS = 2, 16
    NW = NC * NS
    G = NW // B
    CPG = C // G
    NSL = HW // SLICE
    VPS = SLICE // L

    mesh = plsc.VectorSubcoreMesh(core_axis_name="c", subcore_axis_name="s")

    acc_types = [pltpu.VMEM((L * K_SEG,), jnp.float32) for _ in range(CH)]
    buf_types = [pltpu.VMEM((SLICE,), jnp.float32) for _ in range(2 * CH)]
    sem_types = [pltpu.SemaphoreType.DMA for _ in range(2 * CH)]

    @functools.partial(
        pl.kernel,
        out_type=jax.ShapeDtypeStruct((B, C, K_SEG), jnp.float32),
        mesh=mesh,
        scratch_types=[
            pltpu.VMEM((HW,), jnp.int32),
            *buf_types,
            *acc_types,
            pltpu.VMEM((K_SEG,), jnp.float32),
            *sem_types,
        ],
        compiler_params=pltpu.CompilerParams(needs_layout_passes=False),
    )
    def pool(img_hbm, spx_hbm, out_hbm, idx_ref, *rest):
        bufs_a = rest[0:CH]
        bufs_b = rest[CH:2 * CH]
        accs = rest[2 * CH:3 * CH]
        row_ref = rest[3 * CH]
        sems_a = rest[3 * CH + 1:3 * CH + 1 + CH]
        sems_b = rest[3 * CH + 1 + CH:3 * CH + 1 + 2 * CH]
        wid = lax.axis_index("s") * NC + lax.axis_index("c")
        b = wid // G
        g = wid % G

        pltpu.sync_copy(spx_hbm.at[b], idx_ref)
        lane_off = lax.iota(jnp.int32, L) * K_SEG
        neg_inf = jnp.full((L,), -jnp.inf, dtype=jnp.float32)

        def mk_idx(v, carry):
            sl = pl.ds(v * L, L)
            idx_ref[sl] = idx_ref[sl] + lane_off
            return carry

        lax.fori_loop(0, HW // L, mk_idx, 0)

        def init(v, carry):
            sl = pl.ds(v * L, L)
            for acc in accs:
                acc[sl] = neg_inf
            return carry

        lax.fori_loop(0, K_SEG, init, 0)

        def fire(ch0, s, bufs, sems):
            for c in range(CH):
                pltpu.async_copy(
                    img_hbm.at[b, ch0 + c, pl.ds(s * SLICE, SLICE)],
                    bufs[c], sems[c])

        def wait(ch0, bufs, sems):
            for c in range(CH):
                pltpu.make_async_copy(
                    img_hbm.at[b, ch0 + c, pl.ds(0, SLICE)],
                    bufs[c], sems[c]).wait()

        def scatter_slice(bufs, s):
            base = s * SLICE

            def body(v, carry):
                off = v * L
                ivec = idx_ref[pl.ds(base + off, L)]
                dvecs = [bufs[c][pl.ds(off, L)] for c in range(CH)]
                gvs = [plsc.load_gather(accs[c], [ivec])
                       for c in range(CH)]
                for c in range(CH):
                    plsc.store_scatter(accs[c], [ivec],
                                       jnp.maximum(gvs[c], dvecs[c]))
                return carry

            lax.fori_loop(0, VPS, body, 0)

        def per_group(j, carry):
            ch0 = g * CPG + CH * j
            fire(ch0, 0, bufs_a, sems_a)

            def pair(p, c):
                s0 = 2 * p
                wait(ch0, bufs_a, sems_a)
                fire(ch0, s0 + 1, bufs_b, sems_b)
                scatter_slice(bufs_a, s0)
                wait(ch0, bufs_b, sems_b)

                @pl.when(p < NSL // 2 - 1)
                def _():
                    fire(ch0, s0 + 2, bufs_a, sems_a)

                scatter_slice(bufs_b, s0 + 1)
                return c

            lax.fori_loop(0, NSL // 2, pair, 0)

            for c in range(CH):
                def red(kv, cc, acc=accs[c]):
                    vals = []
                    for r in range(L):
                        rsl = pl.ds(r * K_SEG + kv * L, L)
                        vals.append(acc[rsl])
                    for r in range(L):
                        rsl = pl.ds(r * K_SEG + kv * L, L)
                        acc[rsl] = neg_inf
                    while len(vals) > 1:
                        vals = [jnp.maximum(vals[i], vals[i + 1])
                                for i in range(0, len(vals) - 1, 2)] + \
                               ([vals[-1]] if len(vals) % 2 else [])
                    row_ref[pl.ds(kv * L, L)] = vals[0]
                    return cc

                lax.fori_loop(0, K_SEG // L, red, 0)
                pltpu.sync_copy(row_ref, out_hbm.at[b, ch0 + c])
            return carry

        lax.fori_loop(0, CPG // CH, per_group, 0)

    return pool(img, spx)


def kernel(img, spx):
    B, C, H, W = img.shape
    HW = H * W
    img2 = img.reshape(B, C, HW)
    spx2 = spx.reshape(B, HW).astype(jnp.int32)
    return _sup_pix_pool(img2, spx2, B, C, HW)


# speedup vs baseline: 3.0517x; 1.2457x over previous
import functools

import jax
import jax.numpy as jnp
from jax import lax
from jax.experimental import pallas as pl
from jax.experimental.pallas import tpu as pltpu
from jax.experimental.pallas import tpu_sc as plsc

K_SEG = 1024
L = 16
CH = 3
RPS = 8


@functools.partial(jax.jit, static_argnums=(2, 3, 4, 5))
def _sup_pix_pool(img, spx, B, C, H, W):
    HW = H * W
    SLICE = RPS * W
    NC, NS = 2, 16
    NW = NC * NS
    G = NW // B
    CPG = C // G
    NSL = H // RPS
    VPR = W // L

    mesh = plsc.VectorSubcoreMesh(core_axis_name="c", subcore_axis_name="s")

    acc_types = [pltpu.VMEM((L * K_SEG,), jnp.float32) for _ in range(CH)]
    buf_types = [pltpu.VMEM((2 * CH, RPS, W), jnp.float32)]
    sem_types = [pltpu.SemaphoreType.DMA for _ in range(2 * CH)]

    @functools.partial(
        pl.kernel,
        out_type=jax.ShapeDtypeStruct((B, C, K_SEG), jnp.float32),
        mesh=mesh,
        scratch_types=[
            pltpu.VMEM((HW,), jnp.int32),
            *buf_types,
            *acc_types,
            pltpu.VMEM((K_SEG,), jnp.float32),
            *sem_types,
        ],
        compiler_params=pltpu.CompilerParams(needs_layout_passes=False),
    )
    def pool(img_hbm, spx_hbm, out_hbm, idx_ref, *rest):
        big_buf = rest[0]
        bufs_a = [big_buf.at[i] for i in range(CH)]
        bufs_b = [big_buf.at[CH + i] for i in range(CH)]
        accs = rest[1:1 + CH]
        row_ref = rest[1 + CH]
        sems_a = rest[2 + CH:2 + 2 * CH]
        sems_b = rest[2 + 2 * CH:2 + 3 * CH]
        wid = lax.axis_index("s") * NC + lax.axis_index("c")
        b = wid // G
        g = wid % G

        pltpu.sync_copy(spx_hbm.at[b], idx_ref)
        lane_off = lax.iota(jnp.int32, L) * K_SEG
        neg_inf = jnp.full((L,), -jnp.inf, dtype=jnp.float32)

        def mk_idx(v, carry):
            sl = pl.ds(v * L, L)
            idx_ref[sl] = idx_ref[sl] + lane_off
            return carry

        lax.fori_loop(0, HW // L, mk_idx, 0)

        def init(v, carry):
            sl = pl.ds(v * L, L)
            for acc in accs:
                acc[sl] = neg_inf
            return carry

        lax.fori_loop(0, K_SEG, init, 0)

        def fire(ch0, s, bufs, sems):
            for c in range(CH):
                pltpu.async_copy(
                    img_hbm.at[b, ch0 + c, pl.ds(s * RPS, RPS), :],
                    bufs[c], sems[c])

        def wait(ch0, bufs, sems):
            for c in range(CH):
                pltpu.make_async_copy(
                    img_hbm.at[b, ch0 + c, pl.ds(0, RPS), :],
                    bufs[c], sems[c]).wait()

        def scatter_slice(bufs, s):
            def row(r, carry):
                base = s * SLICE + r * W

                def body(q, cc):
                    off = q * L
                    ivec = idx_ref[pl.ds(base + off, L)]
                    dvecs = [bufs[c][r, pl.ds(off, L)] for c in range(CH)]
                    gvs = [plsc.load_gather(accs[c], [ivec])
                           for c in range(CH)]
                    for c in range(CH):
                        plsc.store_scatter(accs[c], [ivec],
                                           jnp.maximum(gvs[c], dvecs[c]))
                    return cc

                lax.fori_loop(0, VPR, body, 0)
                return carry

            lax.fori_loop(0, RPS, row, 0)

        def per_group(j, carry):
            ch0 = g * CPG + CH * j
            fire(ch0, 0, bufs_a, sems_a)

            def pair(p, c):
                s0 = 2 * p
                wait(ch0, bufs_a, sems_a)
                fire(ch0, s0 + 1, bufs_b, sems_b)
                scatter_slice(bufs_a, s0)
                wait(ch0, bufs_b, sems_b)

                @pl.when(p < NSL // 2 - 1)
                def _():
                    fire(ch0, s0 + 2, bufs_a, sems_a)

                scatter_slice(bufs_b, s0 + 1)
                return c

            lax.fori_loop(0, NSL // 2, pair, 0)

            for c in range(CH):
                def red(kv, cc, acc=accs[c]):
                    vals = []
                    for r in range(L):
                        rsl = pl.ds(r * K_SEG + kv * L, L)
                        vals.append(acc[rsl])
                    for r in range(L):
                        rsl = pl.ds(r * K_SEG + kv * L, L)
                        acc[rsl] = neg_inf
                    while len(vals) > 1:
                        vals = [jnp.maximum(vals[i], vals[i + 1])
                                for i in range(0, len(vals) - 1, 2)] + \
                               ([vals[-1]] if len(vals) % 2 else [])
                    row_ref[pl.ds(kv * L, L)] = vals[0]
                    return cc

                lax.fori_loop(0, K_SEG // L, red, 0)
                pltpu.sync_copy(row_ref, out_hbm.at[b, ch0 + c])
            return carry

        lax.fori_loop(0, CPG // CH, per_group, 0)

    return pool(img, spx)


def kernel(img, spx):
    B, C, H, W = img.shape
    spx2 = spx.reshape(B, H * W).astype(jnp.int32)
    return _sup_pix_pool(img, spx2, B, C, H, W)


# speedup vs baseline: 3.3001x; 1.0814x over previous
import functools

import jax
import jax.numpy as jnp
from jax import lax
from jax.experimental import pallas as pl
from jax.experimental.pallas import tpu as pltpu
from jax.experimental.pallas import tpu_sc as plsc

K_SEG = 1024
L = 16
CH = 3
RPS = 8


@functools.partial(jax.jit, static_argnums=(2, 3, 4, 5))
def _sup_pix_pool(img, spx, B, C, H, W):
    HW = H * W
    SLICE = RPS * W
    NC, NS = 2, 16
    NW = NC * NS
    G = NW // B
    CPG = C // G
    NSL = H // RPS
    VPR = W // L

    mesh = plsc.VectorSubcoreMesh(core_axis_name="c", subcore_axis_name="s")

    acc_types = [pltpu.VMEM((L * K_SEG,), jnp.float32) for _ in range(CH)]
    buf_types = [pltpu.VMEM((2 * CH, RPS, W), jnp.float32)]
    sem_types = [pltpu.SemaphoreType.DMA for _ in range(2 * CH)]

    @functools.partial(
        pl.kernel,
        out_type=jax.ShapeDtypeStruct((B, C, K_SEG), jnp.float32),
        mesh=mesh,
        scratch_types=[
            pltpu.VMEM((HW,), jnp.int32),
            *buf_types,
            *acc_types,
            pltpu.VMEM((K_SEG,), jnp.float32),
            *sem_types,
        ],
        compiler_params=pltpu.CompilerParams(needs_layout_passes=False),
    )
    def pool(img_hbm, spx_hbm, out_hbm, idx_ref, *rest):
        big_buf = rest[0]
        bufs_a = [big_buf.at[i] for i in range(CH)]
        bufs_b = [big_buf.at[CH + i] for i in range(CH)]
        accs = rest[1:1 + CH]
        row_ref = rest[1 + CH]
        sems_a = rest[2 + CH:2 + 2 * CH]
        sems_b = rest[2 + 2 * CH:2 + 3 * CH]
        wid = lax.axis_index("s") * NC + lax.axis_index("c")
        b = wid // G
        g = wid % G

        pltpu.sync_copy(spx_hbm.at[b], idx_ref)
        lane_off = lax.iota(jnp.int32, L) * K_SEG
        neg_inf = jnp.full((L,), -jnp.inf, dtype=jnp.float32)

        def mk_idx(v, carry):
            for u in range(4):
                sl = pl.ds(v * (4 * L) + u * L, L)
                idx_ref[sl] = idx_ref[sl] + lane_off
            return carry

        lax.fori_loop(0, HW // (4 * L), mk_idx, 0)

        def init(v, carry):
            sl = pl.ds(v * L, L)
            for acc in accs:
                acc[sl] = neg_inf
            return carry

        lax.fori_loop(0, K_SEG, init, 0)

        def fire(ch0, s, bufs, sems):
            for c in range(CH):
                pltpu.async_copy(
                    img_hbm.at[b, ch0 + c, pl.ds(s * RPS, RPS), :],
                    bufs[c], sems[c])

        def wait(ch0, bufs, sems):
            for c in range(CH):
                pltpu.make_async_copy(
                    img_hbm.at[b, ch0 + c, pl.ds(0, RPS), :],
                    bufs[c], sems[c]).wait()

        def scatter_slice(bufs, s):
            def row(r, carry):
                base = s * SLICE + r * W

                def body(q, cc):
                    for u in range(2):
                        off = q * (2 * L) + u * L
                        ivec = idx_ref[pl.ds(base + off, L)]
                        dvecs = [bufs[c][r, pl.ds(off, L)]
                                 for c in range(CH)]
                        gvs = [plsc.load_gather(accs[c], [ivec])
                               for c in range(CH)]
                        for c in range(CH):
                            plsc.store_scatter(accs[c], [ivec],
                                               jnp.maximum(gvs[c], dvecs[c]))
                    return cc

                lax.fori_loop(0, VPR // 2, body, 0)
                return carry

            lax.fori_loop(0, RPS, row, 0)

        def per_group(j, carry):
            ch0 = g * CPG + CH * j

            def pair(p, c):
                s0 = 2 * p
                wait(ch0, bufs_a, sems_a)
                fire(ch0, s0 + 1, bufs_b, sems_b)
                scatter_slice(bufs_a, s0)
                wait(ch0, bufs_b, sems_b)

                @pl.when(p < NSL // 2 - 1)
                def _():
                    fire(ch0, s0 + 2, bufs_a, sems_a)

                scatter_slice(bufs_b, s0 + 1)
                return c

            lax.fori_loop(0, NSL // 2, pair, 0)

            @pl.when(j < CPG // CH - 1)
            def _():
                fire(ch0 + CH, 0, bufs_a, sems_a)

            for c in range(CH):
                def red(kv, cc, acc=accs[c]):
                    vals = []
                    for r in range(L):
                        rsl = pl.ds(r * K_SEG + kv * L, L)
                        vals.append(acc[rsl])
                    for r in range(L):
                        rsl = pl.ds(r * K_SEG + kv * L, L)
                        acc[rsl] = neg_inf
                    while len(vals) > 1:
                        vals = [jnp.maximum(vals[i], vals[i + 1])
                                for i in range(0, len(vals) - 1, 2)] + \
                               ([vals[-1]] if len(vals) % 2 else [])
                    row_ref[pl.ds(kv * L, L)] = vals[0]
                    return cc

                lax.fori_loop(0, K_SEG // L, red, 0)
                pltpu.sync_copy(row_ref, out_hbm.at[b, ch0 + c])
            return carry

        fire(g * CPG, 0, bufs_a, sems_a)
        lax.fori_loop(0, CPG // CH, per_group, 0)

    return pool(img, spx)


def kernel(img, spx):
    B, C, H, W = img.shape
    spx2 = spx.reshape(B, H * W).astype(jnp.int32)
    return _sup_pix_pool(img, spx2, B, C, H, W)


# speedup vs baseline: 3.3531x; 1.0161x over previous
import functools

import jax
import jax.numpy as jnp
from jax import lax
from jax.experimental import pallas as pl
from jax.experimental.pallas import tpu as pltpu
from jax.experimental.pallas import tpu_sc as plsc

K_SEG = 1024
L = 16
CH = 3
RPS = 8


@functools.partial(jax.jit, static_argnums=(2, 3, 4, 5))
def _sup_pix_pool(img, spx, B, C, H, W):
    HW = H * W
    SLICE = RPS * W
    NC, NS = 2, 16
    NW = NC * NS
    G = NW // B
    CPG = C // G
    NSL = H // RPS
    VPR = W // L

    mesh = plsc.VectorSubcoreMesh(core_axis_name="c", subcore_axis_name="s")

    acc_types = [pltpu.VMEM((L * K_SEG,), jnp.float32) for _ in range(CH)]
    buf_types = [pltpu.VMEM((2 * CH, RPS, W), jnp.float32)]
    sem_types = [pltpu.SemaphoreType.DMA for _ in range(2 * CH)]

    @functools.partial(
        pl.kernel,
        out_type=jax.ShapeDtypeStruct((B, C, K_SEG), jnp.float32),
        mesh=mesh,
        scratch_types=[
            pltpu.VMEM((HW,), jnp.int32),
            *buf_types,
            *acc_types,
            pltpu.VMEM((K_SEG,), jnp.float32),
            *sem_types,
        ],
        compiler_params=pltpu.CompilerParams(needs_layout_passes=False),
    )
    def pool(img_hbm, spx_hbm, out_hbm, idx_ref, *rest):
        big_buf = rest[0]
        bufs_a = [big_buf.at[i] for i in range(CH)]
        bufs_b = [big_buf.at[CH + i] for i in range(CH)]
        accs = rest[1:1 + CH]
        row_ref = rest[1 + CH]
        sems_a = rest[2 + CH:2 + 2 * CH]
        sems_b = rest[2 + 2 * CH:2 + 3 * CH]
        wid = lax.axis_index("s") * NC + lax.axis_index("c")
        b = wid // G
        g = wid % G

        pltpu.sync_copy(spx_hbm.at[b], idx_ref)
        lane_off = lax.iota(jnp.int32, L) * K_SEG
        neg_inf = jnp.full((L,), -jnp.inf, dtype=jnp.float32)

        def mk_idx(v, carry):
            for u in range(4):
                sl = pl.ds(v * (4 * L) + u * L, L)
                idx_ref[sl] = idx_ref[sl] + lane_off
            return carry

        lax.fori_loop(0, HW // (4 * L), mk_idx, 0)

        def init(v, carry):
            sl = pl.ds(v * L, L)
            for acc in accs:
                acc[sl] = neg_inf
            return carry

        lax.fori_loop(0, K_SEG, init, 0)

        def fire(ch0, s, bufs, sems):
            for c in range(CH):
                pltpu.async_copy(
                    img_hbm.at[b, ch0 + c, pl.ds(s * RPS, RPS), :],
                    bufs[c], sems[c])

        def wait(ch0, bufs, sems):
            for c in range(CH):
                pltpu.make_async_copy(
                    img_hbm.at[b, ch0 + c, pl.ds(0, RPS), :],
                    bufs[c], sems[c]).wait()

        def scatter_slice(bufs, s):
            def row(r, carry):
                base = s * SLICE + r * W

                for q in range(VPR):
                    off = q * L
                    ivec = idx_ref[pl.ds(base + off, L)]
                    dvecs = [bufs[c][r, pl.ds(off, L)]
                             for c in range(CH)]
                    gvs = [plsc.load_gather(accs[c], [ivec])
                           for c in range(CH)]
                    for c in range(CH):
                        plsc.store_scatter(accs[c], [ivec],
                                           jnp.maximum(gvs[c], dvecs[c]))
                return carry

            lax.fori_loop(0, RPS, row, 0)

        def per_group(j, carry):
            ch0 = g * CPG + CH * j

            def pair(p, c):
                s0 = 2 * p
                wait(ch0, bufs_a, sems_a)
                fire(ch0, s0 + 1, bufs_b, sems_b)
                scatter_slice(bufs_a, s0)
                wait(ch0, bufs_b, sems_b)

                @pl.when(p < NSL // 2 - 1)
                def _():
                    fire(ch0, s0 + 2, bufs_a, sems_a)

                scatter_slice(bufs_b, s0 + 1)
                return c

            lax.fori_loop(0, NSL // 2, pair, 0)

            @pl.when(j < CPG // CH - 1)
            def _():
                fire(ch0 + CH, 0, bufs_a, sems_a)

            for c in range(CH):
                def red(kv, cc, acc=accs[c]):
                    vals = []
                    for r in range(L):
                        rsl = pl.ds(r * K_SEG + kv * L, L)
                        vals.append(acc[rsl])
                    for r in range(L):
                        rsl = pl.ds(r * K_SEG + kv * L, L)
                        acc[rsl] = neg_inf
                    while len(vals) > 1:
                        vals = [jnp.maximum(vals[i], vals[i + 1])
                                for i in range(0, len(vals) - 1, 2)] + \
                               ([vals[-1]] if len(vals) % 2 else [])
                    row_ref[pl.ds(kv * L, L)] = vals[0]
                    return cc

                lax.fori_loop(0, K_SEG // L, red, 0)
                pltpu.sync_copy(row_ref, out_hbm.at[b, ch0 + c])
            return carry

        fire(g * CPG, 0, bufs_a, sems_a)
        lax.fori_loop(0, CPG // CH, per_group, 0)

    return pool(img, spx)


def kernel(img, spx):
    B, C, H, W = img.shape
    spx2 = spx.reshape(B, H * W).astype(jnp.int32)
    return _sup_pix_pool(img, spx2, B, C, H, W)


# speedup vs baseline: 3.3724x; 1.0057x over previous
import functools

import jax
import jax.numpy as jnp
from jax import lax
from jax.experimental import pallas as pl
from jax.experimental.pallas import tpu as pltpu
from jax.experimental.pallas import tpu_sc as plsc

K_SEG = 1024
L = 16
CH = 3
RPS = 8


@functools.partial(jax.jit, static_argnums=(2, 3, 4, 5))
def _sup_pix_pool(img, spx, B, C, H, W):
    HW = H * W
    SLICE = RPS * W
    NC, NS = 2, 16
    NW = NC * NS
    G = NW // B
    CPG = C // G
    NSL = H // RPS
    VPR = W // L

    mesh = plsc.VectorSubcoreMesh(core_axis_name="c", subcore_axis_name="s")

    acc_types = [pltpu.VMEM((L * K_SEG,), jnp.float32) for _ in range(CH)]
    buf_types = [pltpu.VMEM((2 * CH, RPS, W), jnp.float32)]
    sem_types = [pltpu.SemaphoreType.DMA for _ in range(2 * CH)]

    @functools.partial(
        pl.kernel,
        out_type=jax.ShapeDtypeStruct((B, C, K_SEG), jnp.float32),
        mesh=mesh,
        scratch_types=[
            pltpu.VMEM((HW,), jnp.int32),
            *buf_types,
            *acc_types,
            pltpu.VMEM((K_SEG,), jnp.float32),
            pltpu.VMEM((K_SEG,), jnp.float32),
            *sem_types,
            pltpu.SemaphoreType.DMA,
            pltpu.SemaphoreType.DMA,
        ],
        compiler_params=pltpu.CompilerParams(needs_layout_passes=False),
    )
    def pool(img_hbm, spx_hbm, out_hbm, idx_ref, *rest):
        big_buf = rest[0]
        bufs_a = [big_buf.at[i] for i in range(CH)]
        bufs_b = [big_buf.at[CH + i] for i in range(CH)]
        accs = rest[1:1 + CH]
        row_refs = rest[1 + CH:3 + CH]
        sems_a = rest[3 + CH:3 + 2 * CH]
        sems_b = rest[3 + 2 * CH:3 + 3 * CH]
        sems_r = rest[3 + 3 * CH:5 + 3 * CH]
        wid = lax.axis_index("s") * NC + lax.axis_index("c")
        b = wid // G
        g = wid % G

        pltpu.sync_copy(spx_hbm.at[b], idx_ref)
        lane_off = lax.iota(jnp.int32, L) * K_SEG
        neg_inf = jnp.full((L,), -jnp.inf, dtype=jnp.float32)

        def mk_idx(v, carry):
            for u in range(4):
                sl = pl.ds(v * (4 * L) + u * L, L)
                idx_ref[sl] = idx_ref[sl] + lane_off
            return carry

        lax.fori_loop(0, HW // (4 * L), mk_idx, 0)

        def init(v, carry):
            sl = pl.ds(v * L, L)
            for acc in accs:
                acc[sl] = neg_inf
            return carry

        lax.fori_loop(0, K_SEG, init, 0)

        def fire(ch0, s, bufs, sems):
            for c in range(CH):
                pltpu.async_copy(
                    img_hbm.at[b, ch0 + c, pl.ds(s * RPS, RPS), :],
                    bufs[c], sems[c])

        def wait(ch0, bufs, sems):
            for c in range(CH):
                pltpu.make_async_copy(
                    img_hbm.at[b, ch0 + c, pl.ds(0, RPS), :],
                    bufs[c], sems[c]).wait()

        def scatter_slice(bufs, s):
            def row(r, carry):
                base = s * SLICE + r * W

                for q in range(VPR):
                    off = q * L
                    ivec = idx_ref[pl.ds(base + off, L)]
                    dvecs = [bufs[c][r, pl.ds(off, L)]
                             for c in range(CH)]
                    gvs = [plsc.load_gather(accs[c], [ivec])
                           for c in range(CH)]
                    for c in range(CH):
                        plsc.store_scatter(accs[c], [ivec],
                                           jnp.maximum(gvs[c], dvecs[c]))
                return carry

            lax.fori_loop(0, RPS, row, 0)

        def per_group(j, carry):
            ch0 = g * CPG + CH * j

            def pair(p, c):
                s0 = 2 * p
                wait(ch0, bufs_a, sems_a)
                fire(ch0, s0 + 1, bufs_b, sems_b)
                scatter_slice(bufs_a, s0)
                wait(ch0, bufs_b, sems_b)

                @pl.when(p < NSL // 2 - 1)
                def _():
                    fire(ch0, s0 + 2, bufs_a, sems_a)

                scatter_slice(bufs_b, s0 + 1)
                return c

            lax.fori_loop(0, NSL // 2, pair, 0)

            @pl.when(j < CPG // CH - 1)
            def _():
                fire(ch0 + CH, 0, bufs_a, sems_a)

            for c in range(CH):
                rb = c % 2
                row_ref = row_refs[rb]

                @pl.when((j > 0) | (c >= 2))
                def _(row_ref=row_ref, sem=sems_r[rb]):
                    pltpu.make_async_copy(
                        row_ref, out_hbm.at[b, ch0], sem).wait()

                def red(kv, cc, acc=accs[c], row_ref=row_ref):
                    vals = []
                    for r in range(L):
                        rsl = pl.ds(r * K_SEG + kv * L, L)
                        vals.append(acc[rsl])
                    for r in range(L):
                        rsl = pl.ds(r * K_SEG + kv * L, L)
                        acc[rsl] = neg_inf
                    while len(vals) > 1:
                        vals = [jnp.maximum(vals[i], vals[i + 1])
                                for i in range(0, len(vals) - 1, 2)] + \
                               ([vals[-1]] if len(vals) % 2 else [])
                    row_ref[pl.ds(kv * L, L)] = vals[0]
                    return cc

                lax.fori_loop(0, K_SEG // L, red, 0)
                pltpu.async_copy(row_ref, out_hbm.at[b, ch0 + c],
                                 sems_r[rb])
            return carry

        fire(g * CPG, 0, bufs_a, sems_a)
        lax.fori_loop(0, CPG // CH, per_group, 0)
        for rb in range(2):
            pltpu.make_async_copy(
                row_refs[rb], out_hbm.at[b, 0], sems_r[rb]).wait()

    return pool(img, spx)


def kernel(img, spx):
    B, C, H, W = img.shape
    spx2 = spx.reshape(B, H * W).astype(jnp.int32)
    return _sup_pix_pool(img, spx2, B, C, H, W)
